# initial kernel scaffold (unmeasured)
import jax
import jax.numpy as jnp
from jax import lax
from jax.experimental import pallas as pl
from jax.experimental.pallas import tpu as pltpu

N_DEV = 4
M, N = 8192, 4096
CM = M // N_DEV
SB = 512


def _all_reduce_silu(partial):
    def body(p_ref, o_ref, rbuf, va, vb, rs_send, rs_recv, ag_send,
             ag_recv, cpy_sem):
        i = lax.axis_index("i")
        right = lax.rem(i + 1, N_DEV)
        left = lax.rem(i + N_DEV - 1, N_DEV)

        barrier = pltpu.get_barrier_semaphore()
        for nbr in (left, right):
            pl.semaphore_signal(barrier, inc=1, device_id=(nbr,),
                                device_id_type=pl.DeviceIdType.MESH)
        pl.semaphore_wait(barrier, 2)

        def copy(src, dst):
            c = pltpu.make_async_copy(src, dst, cpy_sem)
            c.start()
            c.wait()

        for s in range(N_DEV - 1):
            sc = lax.rem(i + 2 * N_DEV - 1 - s, N_DEV)
            rc = lax.rem(i + 2 * N_DEV - 2 - s, N_DEV)
            src = p_ref.at[pl.ds(sc * CM, CM)] if s == 0 else rbuf.at[s - 1]
            rdma = pltpu.make_async_remote_copy(
                src_ref=src,
                dst_ref=rbuf.at[s],
                send_sem=rs_send.at[s],
                recv_sem=rs_recv.at[s],
                device_id=(right,),
                device_id_type=pl.DeviceIdType.MESH,
            )
            rdma.start()
            rdma.wait()
            for b in range(CM // SB):
                copy(rbuf.at[s, pl.ds(b * SB, SB)], va)
                copy(p_ref.at[pl.ds(rc * CM + b * SB, SB)], vb)
                if s < N_DEV - 2:
                    va[...] = va[...] + vb[...]
                    copy(va, rbuf.at[s, pl.ds(b * SB, SB)])
                else:
                    acc = va[...] + vb[...]
                    va[...] = acc * jax.nn.sigmoid(acc)
                    copy(va, o_ref.at[pl.ds(i * CM + b * SB, SB)])

        for s in range(N_DEV - 1):
            sc = lax.rem(i + 2 * N_DEV - s, N_DEV)
            rdma = pltpu.make_async_remote_copy(
                src_ref=o_ref.at[pl.ds(sc * CM, CM)],
                dst_ref=o_ref.at[pl.ds(sc * CM, CM)],
                send_sem=ag_send.at[s],
                recv_sem=ag_recv.at[s],
                device_id=(right,),
                device_id_type=pl.DeviceIdType.MESH,
            )
            rdma.start()
            rdma.wait()

    return pl.pallas_call(
        body,
        out_shape=jax.ShapeDtypeStruct((M, N), jnp.float32),
        in_specs=[pl.BlockSpec(memory_space=pltpu.ANY)],
        out_specs=pl.BlockSpec(memory_space=pltpu.ANY),
        scratch_shapes=[
            pltpu.ANY((N_DEV - 1, CM, N), jnp.float32),
            pltpu.VMEM((SB, N), jnp.float32),
            pltpu.VMEM((SB, N), jnp.float32),
            pltpu.SemaphoreType.DMA((N_DEV - 1,)),
            pltpu.SemaphoreType.DMA((N_DEV - 1,)),
            pltpu.SemaphoreType.DMA((N_DEV - 1,)),
            pltpu.SemaphoreType.DMA((N_DEV - 1,)),
            pltpu.SemaphoreType.DMA,
        ],
        compiler_params=pltpu.CompilerParams(collective_id=0),
    )(partial)


def kernel(x, w_mat):
    partial = jnp.dot(x, w_mat, preferred_element_type=jnp.float32)
    return _all_reduce_silu(partial)


# baseline (device time: 2587572 ns/iter reference)
import jax
import jax.numpy as jnp
from jax import lax
from jax.experimental import pallas as pl
from jax.experimental.pallas import tpu as pltpu

N_DEV = 4
M, N = 8192, 4096
CM = M // N_DEV
SB = 512


def _all_reduce_silu(partial):
    def body(p_ref, o_ref, rbuf, va, vb, rs_send, rs_recv, ag_send,
             ag_recv, cpy_sem):
        i = lax.axis_index("i")
        right = lax.rem(i + 1, N_DEV)
        left = lax.rem(i + N_DEV - 1, N_DEV)

        barrier = pltpu.get_barrier_semaphore()
        for nbr in (left, right):
            pl.semaphore_signal(barrier, inc=1, device_id=(nbr,),
                                device_id_type=pl.DeviceIdType.MESH)
        pl.semaphore_wait(barrier, 2)

        def copy(src, dst):
            c = pltpu.make_async_copy(src, dst, cpy_sem)
            c.start()
            c.wait()

        for s in range(N_DEV - 1):
            sc = lax.rem(i + 2 * N_DEV - 1 - s, N_DEV)
            rc = lax.rem(i + 2 * N_DEV - 2 - s, N_DEV)
            src = p_ref.at[pl.ds(sc * CM, CM)] if s == 0 else rbuf.at[s - 1]
            rdma = pltpu.make_async_remote_copy(
                src_ref=src,
                dst_ref=rbuf.at[s],
                send_sem=rs_send.at[s],
                recv_sem=rs_recv.at[s],
                device_id=(right,),
                device_id_type=pl.DeviceIdType.MESH,
            )
            rdma.start()
            rdma.wait()
            for b in range(CM // SB):
                copy(rbuf.at[s, pl.ds(b * SB, SB)], va)
                copy(p_ref.at[pl.ds(rc * CM + b * SB, SB)], vb)
                if s < N_DEV - 2:
                    va[...] = va[...] + vb[...]
                    copy(va, rbuf.at[s, pl.ds(b * SB, SB)])
                else:
                    acc = va[...] + vb[...]
                    va[...] = acc * jax.nn.sigmoid(acc)
                    copy(va, o_ref.at[pl.ds(i * CM + b * SB, SB)])

        for s in range(N_DEV - 1):
            sc = lax.rem(i + 2 * N_DEV - s, N_DEV)
            rdma = pltpu.make_async_remote_copy(
                src_ref=o_ref.at[pl.ds(sc * CM, CM)],
                dst_ref=o_ref.at[pl.ds(sc * CM, CM)],
                send_sem=ag_send.at[s],
                recv_sem=ag_recv.at[s],
                device_id=(right,),
                device_id_type=pl.DeviceIdType.MESH,
            )
            rdma.start()
            rdma.wait()

    out, _ = pl.pallas_call(
        body,
        out_shape=[
            jax.ShapeDtypeStruct((M, N), jnp.float32),
            jax.ShapeDtypeStruct((N_DEV - 1, CM, N), jnp.float32),
        ],
        in_specs=[pl.BlockSpec(memory_space=pl.ANY)],
        out_specs=[
            pl.BlockSpec(memory_space=pl.ANY),
            pl.BlockSpec(memory_space=pl.ANY),
        ],
        scratch_shapes=[
            pltpu.VMEM((SB, N), jnp.float32),
            pltpu.VMEM((SB, N), jnp.float32),
            pltpu.SemaphoreType.DMA((N_DEV - 1,)),
            pltpu.SemaphoreType.DMA((N_DEV - 1,)),
            pltpu.SemaphoreType.DMA((N_DEV - 1,)),
            pltpu.SemaphoreType.DMA((N_DEV - 1,)),
            pltpu.SemaphoreType.DMA,
        ],
        compiler_params=pltpu.CompilerParams(collective_id=0),
    )(partial)
    return out


def kernel(x, w_mat):
    partial = jnp.dot(x, w_mat, preferred_element_type=jnp.float32)
    return _all_reduce_silu(partial)


# device time: 1542092 ns/iter; 1.6780x vs baseline; 1.6780x over previous
import jax
import jax.numpy as jnp
from jax import lax
from jax.experimental import pallas as pl
from jax.experimental.pallas import tpu as pltpu

N_DEV = 4
M, N = 8192, 4096
CM = M // N_DEV
NH = N // 2
SB = 512


def _all_reduce_silu(partial):
    def body(p_ref, o_ref, rbuf, va, vb, rs_send, rs_recv, ag_send,
             ag_recv, cpy_sem):
        i = lax.axis_index("i")
        right = lax.rem(i + 1, N_DEV)
        left = lax.rem(i + N_DEV - 1, N_DEV)

        barrier = pltpu.get_barrier_semaphore()
        for nbr in (left, right):
            pl.semaphore_signal(barrier, inc=1, device_id=(nbr,),
                                device_id_type=pl.DeviceIdType.MESH)
        pl.semaphore_wait(barrier, 2)

        def copy(src, dst):
            c = pltpu.make_async_copy(src, dst, cpy_sem)
            c.start()
            c.wait()

        def rs_rdma(r, s):
            tgt = right if r == 0 else left
            sign = -1 if r == 0 else 1
            sc = lax.rem(i + 2 * N_DEV + sign * (1 + s), N_DEV)
            col = r * NH
            src = (p_ref.at[pl.ds(sc * CM, CM), pl.ds(col, NH)]
                   if s == 0 else rbuf.at[r, s - 1])
            return pltpu.make_async_remote_copy(
                src_ref=src,
                dst_ref=rbuf.at[r, s],
                send_sem=rs_send.at[r, s],
                recv_sem=rs_recv.at[r, s],
                device_id=(tgt,),
                device_id_type=pl.DeviceIdType.MESH,
            )

        def rs_accum(r, s):
            sign = -1 if r == 0 else 1
            rc = lax.rem(i + 2 * N_DEV + sign * (2 + s), N_DEV)
            col = r * NH
            for b in range(CM // SB):
                copy(rbuf.at[r, s, pl.ds(b * SB, SB)], va)
                copy(p_ref.at[pl.ds(rc * CM + b * SB, SB),
                              pl.ds(col, NH)], vb)
                if s < N_DEV - 2:
                    va[...] = va[...] + vb[...]
                    copy(va, rbuf.at[r, s, pl.ds(b * SB, SB)])
                else:
                    acc = va[...] + vb[...]
                    va[...] = acc * jax.nn.sigmoid(acc)
                    copy(va, o_ref.at[pl.ds(i * CM + b * SB, SB),
                                      pl.ds(col, NH)])

        for s in range(N_DEV - 1):
            ra = rs_rdma(0, s)
            rb = rs_rdma(1, s)
            ra.start()
            rb.start()
            ra.wait()
            rs_accum(0, s)
            rb.wait()
            rs_accum(1, s)

        for s in range(N_DEV - 1):
            rdmas = []
            for r in (0, 1):
                tgt = right if r == 0 else left
                sign = -1 if r == 0 else 1
                sc = lax.rem(i + 2 * N_DEV + sign * s, N_DEV)
                col = r * NH
                blk = (pl.ds(sc * CM, CM), pl.ds(col, NH))
                rdmas.append(pltpu.make_async_remote_copy(
                    src_ref=o_ref.at[blk],
                    dst_ref=o_ref.at[blk],
                    send_sem=ag_send.at[r, s],
                    recv_sem=ag_recv.at[r, s],
                    device_id=(tgt,),
                    device_id_type=pl.DeviceIdType.MESH,
                ))
            rdmas[0].start()
            rdmas[1].start()
            rdmas[0].wait()
            rdmas[1].wait()

    out, _ = pl.pallas_call(
        body,
        out_shape=[
            jax.ShapeDtypeStruct((M, N), jnp.float32),
            jax.ShapeDtypeStruct((2, N_DEV - 1, CM, NH), jnp.float32),
        ],
        in_specs=[pl.BlockSpec(memory_space=pl.ANY)],
        out_specs=[
            pl.BlockSpec(memory_space=pl.ANY),
            pl.BlockSpec(memory_space=pl.ANY),
        ],
        scratch_shapes=[
            pltpu.VMEM((SB, NH), jnp.float32),
            pltpu.VMEM((SB, NH), jnp.float32),
            pltpu.SemaphoreType.DMA((2, N_DEV - 1)),
            pltpu.SemaphoreType.DMA((2, N_DEV - 1)),
            pltpu.SemaphoreType.DMA((2, N_DEV - 1)),
            pltpu.SemaphoreType.DMA((2, N_DEV - 1)),
            pltpu.SemaphoreType.DMA,
        ],
        compiler_params=pltpu.CompilerParams(collective_id=0),
    )(partial)
    return out


def kernel(x, w_mat):
    partial = jnp.dot(x, w_mat, preferred_element_type=jnp.float32)
    return _all_reduce_silu(partial)


# device time: 1363141 ns/iter; 1.8982x vs baseline; 1.1313x over previous
import jax
import jax.numpy as jnp
from jax import lax
from jax.experimental import pallas as pl
from jax.experimental.pallas import tpu as pltpu

N_DEV = 4
M, N = 8192, 4096
CM = M // N_DEV
NH = N // 2
SC = 4
SB = CM // SC


def _all_reduce_silu(partial):
    def body(p_ref, o_ref, rbuf, va, vb, rs_send, rs_recv, ag_send,
             ag_recv, cpy_sem):
        i = lax.axis_index("i")
        right = lax.rem(i + 1, N_DEV)
        left = lax.rem(i + N_DEV - 1, N_DEV)

        barrier = pltpu.get_barrier_semaphore()
        for nbr in (left, right):
            pl.semaphore_signal(barrier, inc=1, device_id=(nbr,),
                                device_id_type=pl.DeviceIdType.MESH)
        pl.semaphore_wait(barrier, 2)

        def copy(src, dst):
            c = pltpu.make_async_copy(src, dst, cpy_sem)
            c.start()
            c.wait()

        def rows(c, b):
            return pl.ds(c * CM + b * SB, SB)

        def rs_rdma(r, s, b):
            tgt = right if r == 0 else left
            sign = -1 if r == 0 else 1
            sc = lax.rem(i + 2 * N_DEV + sign * (1 + s), N_DEV)
            src = (p_ref.at[rows(sc, b), pl.ds(r * NH, NH)]
                   if s == 0 else rbuf.at[r, s - 1, pl.ds(b * SB, SB)])
            return pltpu.make_async_remote_copy(
                src_ref=src,
                dst_ref=rbuf.at[r, s, pl.ds(b * SB, SB)],
                send_sem=rs_send.at[r, s, b],
                recv_sem=rs_recv.at[r, s, b],
                device_id=(tgt,),
                device_id_type=pl.DeviceIdType.MESH,
            )

        def ag_rdma(r, s, b):
            tgt = right if r == 0 else left
            sign = -1 if r == 0 else 1
            sc = lax.rem(i + 2 * N_DEV + sign * s, N_DEV)
            blk = (rows(sc, b), pl.ds(r * NH, NH))
            return pltpu.make_async_remote_copy(
                src_ref=o_ref.at[blk],
                dst_ref=o_ref.at[blk],
                send_sem=ag_send.at[r, s, b],
                recv_sem=ag_recv.at[r, s, b],
                device_id=(tgt,),
                device_id_type=pl.DeviceIdType.MESH,
            )

        sent = []

        for b in range(SC):
            for r in (0, 1):
                d = rs_rdma(r, 0, b)
                d.start()
                sent.append(d)

        for s in range(N_DEV - 1):
            for b in range(SC):
                for r in (0, 1):
                    sign = -1 if r == 0 else 1
                    rc = lax.rem(i + 2 * N_DEV + sign * (2 + s), N_DEV)
                    rs_rdma(r, s, b).wait_recv()
                    copy(rbuf.at[r, s, pl.ds(b * SB, SB)], va)
                    copy(p_ref.at[rows(rc, b), pl.ds(r * NH, NH)], vb)
                    if s < N_DEV - 2:
                        va[...] = va[...] + vb[...]
                        copy(va, rbuf.at[r, s, pl.ds(b * SB, SB)])
                        d = rs_rdma(r, s + 1, b)
                        d.start()
                        sent.append(d)
                    else:
                        acc = va[...] + vb[...]
                        va[...] = acc * jax.nn.sigmoid(acc)
                        copy(va, o_ref.at[rows(i, b), pl.ds(r * NH, NH)])
                        d = ag_rdma(r, 0, b)
                        d.start()
                        sent.append(d)

        for s in range(N_DEV - 1):
            for b in range(SC):
                for r in (0, 1):
                    ag_rdma(r, s, b).wait_recv()
                    if s < N_DEV - 2:
                        d = ag_rdma(r, s + 1, b)
                        d.start()
                        sent.append(d)

        for d in sent:
            d.wait_send()

    out, _ = pl.pallas_call(
        body,
        out_shape=[
            jax.ShapeDtypeStruct((M, N), jnp.float32),
            jax.ShapeDtypeStruct((2, N_DEV - 1, CM, NH), jnp.float32),
        ],
        in_specs=[pl.BlockSpec(memory_space=pl.ANY)],
        out_specs=[
            pl.BlockSpec(memory_space=pl.ANY),
            pl.BlockSpec(memory_space=pl.ANY),
        ],
        scratch_shapes=[
            pltpu.VMEM((SB, NH), jnp.float32),
            pltpu.VMEM((SB, NH), jnp.float32),
            pltpu.SemaphoreType.DMA((2, N_DEV - 1, SC)),
            pltpu.SemaphoreType.DMA((2, N_DEV - 1, SC)),
            pltpu.SemaphoreType.DMA((2, N_DEV - 1, SC)),
            pltpu.SemaphoreType.DMA((2, N_DEV - 1, SC)),
            pltpu.SemaphoreType.DMA,
        ],
        compiler_params=pltpu.CompilerParams(collective_id=0),
    )(partial)
    return out


def kernel(x, w_mat):
    partial = jnp.dot(x, w_mat, preferred_element_type=jnp.float32)
    return _all_reduce_silu(partial)
